# baseline (speedup 1.0000x reference)
import functools

import jax
import jax.numpy as jnp
from jax.experimental import pallas as pl
from jax.experimental.pallas import tpu as pltpu

N_E = 1024
E_DIM = 64
BETA = 0.25
LAMBDA_REG = 0.1
UNIFORM_WEIGHT = 0.1

TILE = 4608


def _vq_body(z_ref, emb_ref, zq_ref, idx_ref, loss_ref, qq_ref):
    i = pl.program_id(0)
    z = z_ref[...]
    e = emb_ref[...]

    zz = jnp.sum(z * z, axis=1, keepdims=True)
    ee = jnp.sum(e * e, axis=1)
    two_ze = 2.0 * jax.lax.dot_general(
        z, e, (((1,), (1,)), ((), ())), preferred_element_type=jnp.float32)
    d = (zz + ee[None, :]) - two_ze

    iota = jax.lax.broadcasted_iota(jnp.int32, (TILE, N_E), 1)
    dmin = jnp.min(d, axis=1, keepdims=True)
    idx = jnp.min(jnp.where(d == dmin, iota, N_E), axis=1)
    idx_ref[...] = idx[:, None]

    one_hot = (iota == idx[:, None]).astype(jnp.float32)
    z_q = jax.lax.dot_general(
        one_hot, e, (((1,), (0,)), ((), ())), preferred_element_type=jnp.float32)
    zq_ref[...] = z + (z_q - z)

    diff = z_q - z
    partial = jnp.sum(diff * diff)

    @pl.when(i == 0)
    def _init():
        loss_ref[0, 0] = partial
        sq = (ee[:, None] + ee[None, :]) - 2.0 * jax.lax.dot_general(
            e, e, (((1,), (1,)), ((), ())), preferred_element_type=jnp.float32)
        sq = jnp.maximum(sq, 0.0)
        dist = jnp.where(sq > 0.0, jnp.sqrt(jnp.where(sq > 0.0, sq, 1.0)), 0.0)
        min_d = jnp.min(dist, axis=1)
        max_d = jnp.max(dist, axis=1)
        uniform_loss = jnp.mean(max_d - min_d)
        qq_ref[0, 0] = UNIFORM_WEIGHT * uniform_loss + LAMBDA_REG * jnp.sum(e * e)

    @pl.when(i != 0)
    def _acc():
        loss_ref[0, 0] += partial


@functools.partial(jax.jit, static_argnames=("interpret",))
def _vq_fused(z_flat, emb, interpret=False):
    n = z_flat.shape[0]
    grid = n // TILE
    zq, idx, loss_sum, qq = pl.pallas_call(
        _vq_body,
        grid=(grid,),
        in_specs=[
            pl.BlockSpec((TILE, E_DIM), lambda i: (i, 0)),
            pl.BlockSpec((N_E, E_DIM), lambda i: (0, 0)),
        ],
        out_specs=[
            pl.BlockSpec((TILE, E_DIM), lambda i: (i, 0)),
            pl.BlockSpec((TILE, 1), lambda i: (i, 0)),
            pl.BlockSpec(memory_space=pltpu.SMEM),
            pl.BlockSpec(memory_space=pltpu.SMEM),
        ],
        out_shape=[
            jax.ShapeDtypeStruct((n, E_DIM), jnp.float32),
            jax.ShapeDtypeStruct((n, 1), jnp.int32),
            jax.ShapeDtypeStruct((1, 1), jnp.float32),
            jax.ShapeDtypeStruct((1, 1), jnp.float32),
        ],
        compiler_params=pltpu.CompilerParams(
            dimension_semantics=("arbitrary",)),
        interpret=interpret,
    )(z_flat, emb)
    return zq, idx, loss_sum, qq


def kernel(input, embedding_weight):
    z = input
    z_flat = z.reshape(-1, E_DIM)
    zq, idx, loss_sum, qq = _vq_fused(z_flat, embedding_weight)
    m = loss_sum[0, 0] / (z_flat.shape[0] * E_DIM)
    loss = m + BETA * m
    return (zq.reshape(z.shape), idx, loss, qq[0, 0])


# speedup vs baseline: 1.0080x; 1.0080x over previous
import jax
import jax.numpy as jnp
from jax.experimental import pallas as pl
from jax.experimental.pallas import tpu as pltpu

N_E = 1024
E_DIM = 64
BETA = 0.25
LAMBDA_REG = 0.1
UNIFORM_WEIGHT = 0.1

B = 32
S = 576
N_ROWS = B * S
B_TILE = 8
TILE = B_TILE * S
GRID = B // B_TILE


def _vq_body(z_ref, emb_ref, zq_ref, idx_ref, loss_ref, qq_ref):
    i = pl.program_id(0)
    z = z_ref[...].reshape(TILE, E_DIM)
    e = emb_ref[...]

    zz = jnp.sum(z * z, axis=1, keepdims=True)
    ee = jnp.sum(e * e, axis=1)
    two_ze = 2.0 * jax.lax.dot_general(
        z, e, (((1,), (1,)), ((), ())), preferred_element_type=jnp.float32)
    d = (zz + ee[None, :]) - two_ze

    iota = jax.lax.broadcasted_iota(jnp.int32, (TILE, N_E), 1)
    dmin = jnp.min(d, axis=1, keepdims=True)
    idx = jnp.min(jnp.where(d == dmin, iota, N_E), axis=1)
    idx_ref[...] = idx[:, None]

    one_hot = (iota == idx[:, None]).astype(jnp.float32)
    z_q = jax.lax.dot_general(
        one_hot, e, (((1,), (0,)), ((), ())), preferred_element_type=jnp.float32)
    zq_ref[...] = (z + (z_q - z)).reshape(B_TILE, S, E_DIM)

    diff = z_q - z
    partial = jnp.sum(diff * diff)

    @pl.when(i == 0)
    def _init():
        loss_ref[0, 0] = partial
        sq = (ee[:, None] + ee[None, :]) - 2.0 * jax.lax.dot_general(
            e, e, (((1,), (1,)), ((), ())), preferred_element_type=jnp.float32)
        sq = jnp.maximum(sq, 0.0)
        dist = jnp.where(sq > 0.0, jnp.sqrt(jnp.where(sq > 0.0, sq, 1.0)), 0.0)
        min_d = jnp.min(dist, axis=1)
        max_d = jnp.max(dist, axis=1)
        uniform_loss = jnp.mean(max_d - min_d)
        qq_ref[0, 0] = UNIFORM_WEIGHT * uniform_loss + LAMBDA_REG * jnp.sum(e * e)

    @pl.when(i != 0)
    def _acc():
        loss_ref[0, 0] += partial

    @pl.when(i == GRID - 1)
    def _final():
        m = loss_ref[0, 0] / (N_ROWS * E_DIM)
        loss_ref[0, 0] = m + BETA * m


def kernel(input, embedding_weight):
    zq, idx, loss, qq = pl.pallas_call(
        _vq_body,
        grid=(GRID,),
        in_specs=[
            pl.BlockSpec((B_TILE, S, E_DIM), lambda i: (i, 0, 0)),
            pl.BlockSpec((N_E, E_DIM), lambda i: (0, 0)),
        ],
        out_specs=[
            pl.BlockSpec((B_TILE, S, E_DIM), lambda i: (i, 0, 0)),
            pl.BlockSpec((TILE, 1), lambda i: (i, 0)),
            pl.BlockSpec(memory_space=pltpu.SMEM),
            pl.BlockSpec(memory_space=pltpu.SMEM),
        ],
        out_shape=[
            jax.ShapeDtypeStruct((B, S, E_DIM), jnp.float32),
            jax.ShapeDtypeStruct((N_ROWS, 1), jnp.int32),
            jax.ShapeDtypeStruct((1, 1), jnp.float32),
            jax.ShapeDtypeStruct((1, 1), jnp.float32),
        ],
        compiler_params=pltpu.CompilerParams(
            dimension_semantics=("arbitrary",)),
    )(input, embedding_weight)
    return (zq, idx, loss[0, 0], qq[0, 0])


# speedup vs baseline: 1.0081x; 1.0001x over previous
import jax
import jax.numpy as jnp
from jax.experimental import pallas as pl
from jax.experimental.pallas import tpu as pltpu

N_E = 1024
E_DIM = 64
BETA = 0.25
LAMBDA_REG = 0.1
UNIFORM_WEIGHT = 0.1

B = 32
S = 576
N_ROWS = B * S
B_TILE = 8
TILE = B_TILE * S
GRID = B // B_TILE


def _vq_body(z_ref, emb_ref, zq_ref, idx_ref, loss_ref, qq_ref):
    i = pl.program_id(0)
    z = z_ref[...].reshape(TILE, E_DIM)
    e = emb_ref[...]

    zz = jnp.sum(z * z, axis=1, keepdims=True)
    ee = jnp.sum(e * e, axis=1)
    two_ze = 2.0 * jax.lax.dot_general(
        z, e, (((1,), (1,)), ((), ())), preferred_element_type=jnp.float32)
    d = (zz + ee[None, :]) - two_ze

    iota = jax.lax.broadcasted_iota(jnp.int32, (TILE, N_E), 1)
    dmin = jnp.min(d, axis=1, keepdims=True)
    idx = jnp.min(jnp.where(d == dmin, iota, N_E), axis=1)
    idx_ref[...] = idx[:, None]

    one_hot = (iota == idx[:, None]).astype(jnp.bfloat16)
    z_q = jax.lax.dot_general(
        one_hot, e.astype(jnp.bfloat16), (((1,), (0,)), ((), ())),
        preferred_element_type=jnp.float32)
    zq_ref[...] = (z + (z_q - z)).reshape(B_TILE, S, E_DIM)

    diff = z_q - z
    partial = jnp.sum(diff * diff)

    @pl.when(i == 0)
    def _init():
        loss_ref[0, 0] = partial
        sq = (ee[:, None] + ee[None, :]) - 2.0 * jax.lax.dot_general(
            e, e, (((1,), (1,)), ((), ())), preferred_element_type=jnp.float32)
        sq = jnp.maximum(sq, 0.0)
        dist = jnp.where(sq > 0.0, jnp.sqrt(jnp.where(sq > 0.0, sq, 1.0)), 0.0)
        min_d = jnp.min(dist, axis=1)
        max_d = jnp.max(dist, axis=1)
        uniform_loss = jnp.mean(max_d - min_d)
        qq_ref[0, 0] = UNIFORM_WEIGHT * uniform_loss + LAMBDA_REG * jnp.sum(e * e)

    @pl.when(i != 0)
    def _acc():
        loss_ref[0, 0] += partial

    @pl.when(i == GRID - 1)
    def _final():
        m = loss_ref[0, 0] / (N_ROWS * E_DIM)
        loss_ref[0, 0] = m + BETA * m


def kernel(input, embedding_weight):
    zq, idx, loss, qq = pl.pallas_call(
        _vq_body,
        grid=(GRID,),
        in_specs=[
            pl.BlockSpec((B_TILE, S, E_DIM), lambda i: (i, 0, 0)),
            pl.BlockSpec((N_E, E_DIM), lambda i: (0, 0)),
        ],
        out_specs=[
            pl.BlockSpec((B_TILE, S, E_DIM), lambda i: (i, 0, 0)),
            pl.BlockSpec((TILE, 1), lambda i: (i, 0)),
            pl.BlockSpec(memory_space=pltpu.SMEM),
            pl.BlockSpec(memory_space=pltpu.SMEM),
        ],
        out_shape=[
            jax.ShapeDtypeStruct((B, S, E_DIM), jnp.float32),
            jax.ShapeDtypeStruct((N_ROWS, 1), jnp.int32),
            jax.ShapeDtypeStruct((1, 1), jnp.float32),
            jax.ShapeDtypeStruct((1, 1), jnp.float32),
        ],
        compiler_params=pltpu.CompilerParams(
            dimension_semantics=("arbitrary",)),
    )(input, embedding_weight)
    return (zq, idx, loss[0, 0], qq[0, 0])
